# Initial kernel scaffold; baseline (speedup 1.0000x reference)
import jax
import jax.numpy as jnp
from jax.experimental import pallas as pl


def kernel(positions, entity_type, keys_table, values_table):
    raise NotImplementedError("write your pallas kernel here")


# speedup vs baseline: 2.3734x; 2.3734x over previous
import functools

import jax
import jax.numpy as jnp
from jax import lax
from jax.experimental import pallas as pl
from jax.experimental.pallas import tpu as pltpu
from jax.experimental.pallas import tpu_sc as plsc

_EXT = 10.0
_NPOS = 441
_B, _S, _D = 4, 256, 64
_NPAIR = _B * _S

_info = plsc.get_sparse_core_info()
_NC, _NS, _NL = _info.num_cores, _info.num_subcores, _info.num_lanes
_NW = _NC * _NS
_ROWS_PER_W = _NPAIR // _NW
_W_PER_B = _NW // _B


def _bucket(d):
    a = jnp.minimum(jnp.abs(d), jnp.float32(_EXT))
    a5 = a + jnp.float32(0.5)
    ti = a5.astype(jnp.int32)
    tie = ti.astype(jnp.float32) == a5
    odd = jnp.bitwise_and(ti, 1) == 1
    r = ti - jnp.where(jnp.logical_and(tie, odd), 1, 0)
    r = jnp.where(d < jnp.float32(0.0), -r, r)
    return r + 10


def _sc_body(px_hbm, py_hbm, et_hbm, keys_hbm, vals_hbm, outk_hbm, outv_hbm,
             px_v, py_v, vt_v, idx_v, vdx_v, krows, vrows, sem):
    wid = lax.axis_index("s") * _NC + lax.axis_index("c")
    b = wid // _W_PER_B
    i_base = (wid % _W_PER_B) * _ROWS_PER_W

    pltpu.sync_copy(px_hbm.at[b], px_v)
    pltpu.sync_copy(py_hbm.at[b], py_v)
    pltpu.sync_copy(et_hbm.at[b], vt_v)
    for jj in range(_S // _NL):
        vt_v[jj] = vt_v[jj] * _NPOS

    lanes = lax.iota(jnp.int32, _NL)

    def row_step(p, carry):
        i = i_base + p
        l = i % _NL
        qx = px_v[i // _NL]
        qy = py_v[i // _NL]
        li = jnp.full((_NL, 1), l, jnp.int32)
        dnums = lax.GatherDimensionNumbers(
            offset_dims=(), collapsed_slice_dims=(0,), start_index_map=(0,))
        xi = lax.gather(qx, li, dnums, (1,),
                        mode=lax.GatherScatterMode.PROMISE_IN_BOUNDS)
        yi = lax.gather(qy, li, dnums, (1,),
                        mode=lax.GatherScatterMode.PROMISE_IN_BOUNDS)
        for jj in range(_S // _NL):
            idx = _bucket(px_v[jj] - xi) + 21 * _bucket(py_v[jj] - yi)
            h, o = divmod(jj * _NL, 128)
            idx_v[h, pl.ds(o, _NL)] = idx
            vdx_v[h, pl.ds(o, _NL)] = idx + vt_v[jj]
        cps = []
        for h in range(2):
            dst = pl.ds(h * 128, 128)
            cps.append(pltpu.async_copy(keys_hbm.at[idx_v.at[h]], krows.at[dst], sem))
            cps.append(pltpu.async_copy(vals_hbm.at[vdx_v.at[h]], vrows.at[dst], sem))
        for c in cps:
            c.wait()
        base = (b * _S + i) * _S
        pltpu.sync_copy(krows, outk_hbm.at[pl.ds(base, _S)])
        pltpu.sync_copy(vrows, outv_hbm.at[pl.ds(base, _S)])
        return carry

    lax.fori_loop(0, _ROWS_PER_W, row_step, 0)


@functools.partial(jax.jit, static_argnums=())
def _sc_call(px, py, et, keys_table, values_table):
    mesh = plsc.VectorSubcoreMesh(core_axis_name="c", subcore_axis_name="s")
    f32 = jnp.float32
    run = pl.kernel(
        _sc_body,
        mesh=mesh,
        compiler_params=pltpu.CompilerParams(use_tc_tiling_on_sc=False),
        out_type=[
            jax.ShapeDtypeStruct((_NPAIR * _S, _D), f32),
            jax.ShapeDtypeStruct((_NPAIR * _S, _D), f32),
        ],
        scratch_types=[
            pltpu.VMEM((_S // _NL, _NL), f32),
            pltpu.VMEM((_S // _NL, _NL), f32),
            pltpu.VMEM((_S // _NL, _NL), jnp.int32),
            pltpu.VMEM((2, 128), jnp.int32),
            pltpu.VMEM((2, 128), jnp.int32),
            pltpu.VMEM((_S, _D), f32),
            pltpu.VMEM((_S, _D), f32),
            pltpu.SemaphoreType.DMA,
        ],
    )
    return run(px, py, et, keys_table, values_table)


def kernel(positions, entity_type, keys_table, values_table):
    px = positions[..., 0].reshape(_B, _S // _NL, _NL)
    py = positions[..., 1].reshape(_B, _S // _NL, _NL)
    et = entity_type.astype(jnp.int32).reshape(_B, _S // _NL, _NL)
    outk, outv = _sc_call(px, py, et, keys_table, values_table)
    return (outk.reshape(_B, _S, _S, _D), outv.reshape(_B, _S, _S, _D))


# speedup vs baseline: 5.1298x; 2.1614x over previous
import functools

import jax
import jax.numpy as jnp
from jax import lax
from jax.experimental import pallas as pl
from jax.experimental.pallas import tpu as pltpu
from jax.experimental.pallas import tpu_sc as plsc

_EXT = 10.0
_NPOS = 441
_NENT = 4
_B, _S, _D = 4, 256, 64
_NPAIR = _B * _S

_info = plsc.get_sparse_core_info()
_NC, _NS, _NL = _info.num_cores, _info.num_subcores, _info.num_lanes
_NW = _NC * _NS
_ROWS_PER_W = _NPAIR // _NW
_W_PER_B = _NW // _B


def _bucket(d):
    a = jnp.minimum(jnp.abs(d), jnp.float32(_EXT))
    a5 = a + jnp.float32(0.5)
    ti = a5.astype(jnp.int32)
    tie = ti.astype(jnp.float32) == a5
    odd = jnp.bitwise_and(ti, 1) == 1
    r = ti - jnp.where(jnp.logical_and(tie, odd), 1, 0)
    r = jnp.where(d < jnp.float32(0.0), -r, r)
    return r + 10


def _sc_body(px_hbm, py_hbm, et_hbm, keys_hbm, vals_hbm, outk_hbm, outv_hbm,
             keys_sh, vals_sh, px_v, py_v, vt_v, idx_v, vdx_v, krows, vrows,
             sem_g, sem_o):
    cid = lax.axis_index("c")
    sid = lax.axis_index("s")
    wid = sid * _NC + cid
    b = wid // _W_PER_B
    i_base = (wid % _W_PER_B) * _ROWS_PER_W

    @pl.when(sid == 0)
    def _stage():
        pltpu.sync_copy(keys_hbm, keys_sh)
        pltpu.sync_copy(vals_hbm, vals_sh)

    pltpu.sync_copy(px_hbm.at[b], px_v)
    pltpu.sync_copy(py_hbm.at[b], py_v)
    pltpu.sync_copy(et_hbm.at[b], vt_v)
    for jj in range(_S // _NL):
        vt_v[jj] = vt_v[jj] * _NPOS

    plsc.subcore_barrier()

    dnums = lax.GatherDimensionNumbers(
        offset_dims=(), collapsed_slice_dims=(0,), start_index_map=(0,))

    def compute_idx(p, s):
        i = i_base + p
        l = i % _NL
        qx = px_v[i // _NL]
        qy = py_v[i // _NL]
        li = jnp.full((_NL, 1), l, jnp.int32)
        xi = lax.gather(qx, li, dnums, (1,),
                        mode=lax.GatherScatterMode.PROMISE_IN_BOUNDS)
        yi = lax.gather(qy, li, dnums, (1,),
                        mode=lax.GatherScatterMode.PROMISE_IN_BOUNDS)
        for jj in range(_S // _NL):
            idx = _bucket(px_v[jj] - xi) + 21 * _bucket(py_v[jj] - yi)
            h, o = divmod(jj * _NL, 128)
            idx_v[s, h, pl.ds(o, _NL)] = idx
            vdx_v[s, h, pl.ds(o, _NL)] = idx + vt_v[jj]

    def gather_copies(s):
        cps = []
        for h in range(2):
            dst = pl.ds(h * 128, 128)
            cps.append(pltpu.make_async_copy(
                keys_sh.at[idx_v.at[s].at[h]], krows.at[s].at[dst], sem_g))
            cps.append(pltpu.make_async_copy(
                vals_sh.at[vdx_v.at[s].at[h]], vrows.at[s].at[dst], sem_g))
        return cps

    def out_copies(p, s):
        base = (b * _S + (i_base + p)) * _S
        return [
            pltpu.make_async_copy(krows.at[s], outk_hbm.at[pl.ds(base, _S)], sem_o),
            pltpu.make_async_copy(vrows.at[s], outv_hbm.at[pl.ds(base, _S)], sem_o),
        ]

    compute_idx(0, 0)
    for c in gather_copies(0):
        c.start()

    def step(g, carry):
        for s in range(2):
            p = 2 * g + s
            sn = 1 - s
            for c in gather_copies(s):
                c.wait()

            @pl.when(p >= 1)
            def _wait_prev_out():
                for c in out_copies(p - 1, sn):
                    c.wait()

            for c in out_copies(p, s):
                c.start()

            @pl.when(p + 1 < _ROWS_PER_W)
            def _prefetch_next():
                compute_idx(p + 1, sn)
                for c in gather_copies(sn):
                    c.start()
        return carry

    lax.fori_loop(0, _ROWS_PER_W // 2, step, 0)
    for c in out_copies(_ROWS_PER_W - 1, 1):
        c.wait()


@functools.partial(jax.jit, static_argnums=())
def _sc_call(px, py, et, keys_table, values_table):
    mesh = plsc.VectorSubcoreMesh(core_axis_name="c", subcore_axis_name="s")
    f32 = jnp.float32
    run = pl.kernel(
        _sc_body,
        mesh=mesh,
        compiler_params=pltpu.CompilerParams(use_tc_tiling_on_sc=False),
        out_type=[
            jax.ShapeDtypeStruct((_NPAIR * _S, _D), f32),
            jax.ShapeDtypeStruct((_NPAIR * _S, _D), f32),
        ],
        scratch_types=[
            pltpu.VMEM_SHARED((_NPOS, _D), f32),
            pltpu.VMEM_SHARED((_NPOS * _NENT, _D), f32),
            pltpu.VMEM((_S // _NL, _NL), f32),
            pltpu.VMEM((_S // _NL, _NL), f32),
            pltpu.VMEM((_S // _NL, _NL), jnp.int32),
            pltpu.VMEM((2, 2, 128), jnp.int32),
            pltpu.VMEM((2, 2, 128), jnp.int32),
            pltpu.VMEM((2, _S, _D), f32),
            pltpu.VMEM((2, _S, _D), f32),
            pltpu.SemaphoreType.DMA,
            pltpu.SemaphoreType.DMA,
        ],
    )
    return run(px, py, et, keys_table, values_table)


def kernel(positions, entity_type, keys_table, values_table):
    px = positions[..., 0].reshape(_B, _S // _NL, _NL)
    py = positions[..., 1].reshape(_B, _S // _NL, _NL)
    et = entity_type.astype(jnp.int32).reshape(_B, _S // _NL, _NL)
    outk, outv = _sc_call(px, py, et, keys_table, values_table)
    return (outk.reshape(_B, _S, _S, _D), outv.reshape(_B, _S, _S, _D))


# speedup vs baseline: 7.4244x; 1.4473x over previous
import functools

import jax
import jax.numpy as jnp
from jax import lax
from jax.experimental import pallas as pl
from jax.experimental.pallas import tpu as pltpu
from jax.experimental.pallas import tpu_sc as plsc

_EXT = 10.0
_NPOS = 441
_NENT = 4
_B, _S, _D = 4, 256, 64
_NPAIR = _B * _S

_info = plsc.get_sparse_core_info()
_NC, _NS, _NL = _info.num_cores, _info.num_subcores, _info.num_lanes
_NW = _NC * _NS
_ROWS_PER_W = _NPAIR // _NW
_W_PER_B = _NW // _B


def _bucket(d):
    a = jnp.minimum(jnp.abs(d), jnp.float32(_EXT))
    a5 = a + jnp.float32(0.5)
    ti = a5.astype(jnp.int32)
    tie = ti.astype(jnp.float32) == a5
    odd = jnp.bitwise_and(ti, 1) == 1
    r = ti - jnp.where(jnp.logical_and(tie, odd), 1, 0)
    r = jnp.where(d < jnp.float32(0.0), -r, r)
    return r + 10


def _sc_body(px_hbm, py_hbm, et_hbm, keys_hbm, vals_hbm, outk_hbm, outv_hbm,
             keys_sh, vals_sh, px_v, py_v, vt_v, idx_v, vdx_v, krows, vrows,
             sem_g, sem_o):
    cid = lax.axis_index("c")
    sid = lax.axis_index("s")
    wid = sid * _NC + cid
    b = wid // _W_PER_B
    i_base = (wid % _W_PER_B) * _ROWS_PER_W

    @pl.when(sid == 0)
    def _stage():
        pltpu.sync_copy(keys_hbm, keys_sh)
        pltpu.sync_copy(vals_hbm, vals_sh)

    pltpu.sync_copy(px_hbm.at[b], px_v)
    pltpu.sync_copy(py_hbm.at[b], py_v)
    pltpu.sync_copy(et_hbm.at[b], vt_v)
    for jj in range(_S // _NL):
        vt_v[jj] = vt_v[jj] * _NPOS

    plsc.subcore_barrier()

    dnums = lax.GatherDimensionNumbers(
        offset_dims=(), collapsed_slice_dims=(0,), start_index_map=(0,))

    def compute_idx(p, s):
        i = i_base + p
        u = jnp.where(i < _S // 2, 2 * i, 2 * i - (_S - 1))
        l = u % _NL
        qx = px_v[u // _NL]
        qy = py_v[u // _NL]
        li = jnp.full((_NL, 1), l, jnp.int32)
        xi = lax.gather(qx, li, dnums, (1,),
                        mode=lax.GatherScatterMode.PROMISE_IN_BOUNDS)
        yi = lax.gather(qy, li, dnums, (1,),
                        mode=lax.GatherScatterMode.PROMISE_IN_BOUNDS)
        for jj in range(_S // _NL):
            idx = _bucket(px_v[jj] - xi) + 21 * _bucket(py_v[jj] - yi)
            h, o = divmod(jj * _NL, 128)
            idx_v[s, h, pl.ds(o, _NL)] = idx
            vdx_v[s, h, pl.ds(o, _NL)] = idx + vt_v[jj]

    def gather_copies(s):
        cps = []
        for h in range(2):
            dst = pl.ds(h * 128, 128)
            cps.append(pltpu.make_async_copy(
                keys_sh.at[idx_v.at[s].at[h]], krows.at[s].at[dst], sem_g))
            cps.append(pltpu.make_async_copy(
                vals_sh.at[vdx_v.at[s].at[h]], vrows.at[s].at[dst], sem_g))
        return cps

    def out_copies(p, s):
        base = (b * _S + (i_base + p)) * _S
        return [
            pltpu.make_async_copy(krows.at[s], outk_hbm.at[pl.ds(base, _S)], sem_o),
            pltpu.make_async_copy(vrows.at[s], outv_hbm.at[pl.ds(base, _S)], sem_o),
        ]

    compute_idx(0, 0)
    for c in gather_copies(0):
        c.start()

    def step(g, carry):
        for s in range(2):
            p = 2 * g + s
            sn = 1 - s
            for c in gather_copies(s):
                c.wait()

            @pl.when(p >= 1)
            def _wait_prev_out():
                for c in out_copies(p - 1, sn):
                    c.wait()

            for c in out_copies(p, s):
                c.start()

            @pl.when(p + 1 < _ROWS_PER_W)
            def _prefetch_next():
                compute_idx(p + 1, sn)
                for c in gather_copies(sn):
                    c.start()
        return carry

    lax.fori_loop(0, _ROWS_PER_W // 2, step, 0)
    for c in out_copies(_ROWS_PER_W - 1, 1):
        c.wait()


@functools.partial(jax.jit, static_argnums=())
def _sc_call(px, py, et, keys_table, values_table):
    mesh = plsc.VectorSubcoreMesh(core_axis_name="c", subcore_axis_name="s")
    f32 = jnp.float32
    run = pl.kernel(
        _sc_body,
        mesh=mesh,
        compiler_params=pltpu.CompilerParams(use_tc_tiling_on_sc=False),
        out_type=[
            jax.ShapeDtypeStruct((_NPAIR * _S, _D), f32),
            jax.ShapeDtypeStruct((_NPAIR * _S, _D), f32),
        ],
        scratch_types=[
            pltpu.VMEM_SHARED((_NPOS, _D), f32),
            pltpu.VMEM_SHARED((_NPOS * _NENT, _D), f32),
            pltpu.VMEM((_S // _NL, _NL), f32),
            pltpu.VMEM((_S // _NL, _NL), f32),
            pltpu.VMEM((_S // _NL, _NL), jnp.int32),
            pltpu.VMEM((2, 2, 128), jnp.int32),
            pltpu.VMEM((2, 2, 128), jnp.int32),
            pltpu.VMEM((2, _S, _D), f32),
            pltpu.VMEM((2, _S, _D), f32),
            pltpu.SemaphoreType.DMA,
            pltpu.SemaphoreType.DMA,
        ],
    )
    return run(px, py, et, keys_table, values_table)


_TG = 8


def _tc_tx_body(xk_ref, xv_ref, yk_ref, yv_ref):
    for ref, out in ((xk_ref, yk_ref), (xv_ref, yv_ref)):
        x3 = ref[...].reshape(_TG, 128, 128)
        lo = jnp.swapaxes(x3[:, :, :_D], 1, 2)
        hi = jnp.swapaxes(x3[:, :, _D:], 1, 2)
        out[...] = jnp.concatenate([lo, hi], axis=2)


@jax.jit
def _tc_transpose(xk, xv):
    n = _NPAIR // _TG
    blk_in = pl.BlockSpec((_TG * 128, 128), lambda p: (p, 0))
    blk_out = pl.BlockSpec((_TG, _D, _S), lambda p: (p, 0, 0))
    return pl.pallas_call(
        _tc_tx_body,
        grid=(n,),
        in_specs=[blk_in, blk_in],
        out_specs=[blk_out, blk_out],
        out_shape=[
            jax.ShapeDtypeStruct((_NPAIR, _D, _S), jnp.float32),
            jax.ShapeDtypeStruct((_NPAIR, _D, _S), jnp.float32),
        ],
    )(xk, xv)


def _interleave_j(a):
    return jnp.stack([a[..., : _S // 2], a[..., _S // 2:]], axis=-1).reshape(
        *a.shape[:-1], _S)


def kernel(positions, entity_type, keys_table, values_table):
    px = _interleave_j(positions[..., 0]).reshape(_B, _S // _NL, _NL)
    py = _interleave_j(positions[..., 1]).reshape(_B, _S // _NL, _NL)
    et = _interleave_j(entity_type.astype(jnp.int32)).reshape(_B, _S // _NL, _NL)
    outk, outv = _sc_call(px, py, et, keys_table, values_table)
    tk, tv = _tc_transpose(outk.reshape(_NPAIR * 128, 128),
                           outv.reshape(_NPAIR * 128, 128))
    tk = tk.reshape(_B, _S, _D, _S).transpose(0, 1, 3, 2)
    tv = tv.reshape(_B, _S, _D, _S).transpose(0, 1, 3, 2)
    return (tk, tv)
